# baseline (speedup 1.0000x reference)
import jax
import jax.numpy as jnp
from jax.experimental import pallas as pl

_TS = 1024


def _add_kernel(x_ref, pos_ref, o_ref):
    o_ref[...] = x_ref[...] + pos_ref[...]


def kernel(x, pos_table):
    B, T, E = x.shape
    grid = (T // _TS, B)
    return pl.pallas_call(
        _add_kernel,
        grid=grid,
        in_specs=[
            pl.BlockSpec((1, _TS, E), lambda t, b: (b, t, 0)),
            pl.BlockSpec((_TS, E), lambda t, b: (t, 0)),
        ],
        out_specs=pl.BlockSpec((1, _TS, E), lambda t, b: (b, t, 0)),
        out_shape=jax.ShapeDtypeStruct((B, T, E), x.dtype),
    )(x, pos_table)
